# baseline (speedup 1.0000x reference)
import functools

import jax
import jax.numpy as jnp
from jax import lax
from jax.experimental import pallas as pl
from jax.experimental.pallas import tpu as pltpu
from jax.experimental.pallas import tpu_sc as plsc


def _make_sc_gather(V, D, N):
    info = plsc.get_sparse_core_info()
    NW = info.num_cores * info.num_subcores
    b_per_w = N // NW
    assert N % NW == 0 and b_per_w % 8 == 0 and D % info.num_lanes == 0
    mesh = plsc.VectorSubcoreMesh(core_axis_name="c", subcore_axis_name="s")

    @functools.partial(
        pl.kernel,
        mesh=mesh,
        out_type=jax.ShapeDtypeStruct((N, D), jnp.float32),
        scratch_types=[
            pltpu.VMEM((b_per_w,), jnp.int32),
            pltpu.VMEM((b_per_w, D), jnp.float32),
            pltpu.SemaphoreType.DMA,
        ],
    )
    def gather_k(table_hbm, idx_hbm, out_hbm, idx_v, rows_v, sem):
        wid = lax.axis_index("s") * info.num_cores + lax.axis_index("c")
        base = wid * b_per_w
        pltpu.sync_copy(idx_hbm.at[pl.ds(base, b_per_w)], idx_v)
        pltpu.async_copy(table_hbm.at[idx_v], rows_v, sem).wait()
        pltpu.sync_copy(rows_v, out_hbm.at[pl.ds(base, b_per_w)])

    return gather_k


def _prep_body(B, S, H, R, E,
               tok_ref, pos_ref, g_ref, be_ref, Wp_ref, bp_ref, Wr_ref,
               br_ref, hn_ref, w_ref, i_ref):
    tok = tok_ref[...]
    h = tok + pos_ref[...][None, :, :]
    mu = jnp.mean(h, axis=-1, keepdims=True)
    var = jnp.mean((h - mu) ** 2, axis=-1, keepdims=True)
    hn = (h - mu) * lax.rsqrt(var + 1e-5) * g_ref[...] + be_ref[...]
    hn_ref[...] = hn
    pooled = jnp.mean(hn, axis=1)
    r = lax.dot_general(pooled, Wp_ref[...], (((1,), (1,)), ((), ())),
                        preferred_element_type=jnp.float32) + bp_ref[...]
    logits = lax.dot_general(r, Wr_ref[...], (((1,), (1,)), ((), ())),
                             preferred_element_type=jnp.float32) + br_ref[...]
    m = jnp.max(logits, axis=1, keepdims=True)
    ex = jnp.exp(logits - m)
    p = ex / jnp.sum(ex, axis=1, keepdims=True)
    idx = lax.broadcasted_iota(jnp.int32, (B, E), 1)
    p1 = jnp.max(p, axis=1, keepdims=True)
    i1 = jnp.min(jnp.where(p == p1, idx, E), axis=1, keepdims=True)
    pm = jnp.where(idx == i1, -1.0, p)
    p2 = jnp.max(pm, axis=1, keepdims=True)
    i2 = jnp.min(jnp.where(pm == p2, idx, E), axis=1, keepdims=True)
    denom = p1 + p2 + 1e-8
    w_ref[...] = jnp.concatenate([p1, p2], axis=1) / denom
    i_ref[...] = jnp.concatenate([i1, i2], axis=1)


def _make_moe_logits_body(B, S, H, I, K, N, NI):
    NMOE = B * K * NI

    def body(ids_ref, wts_ref, hn_ref, wg_ref, wu_ref, wd_ref, emb_ref,
             out_ref, comb_ref, comb16_ref):
        s = pl.program_id(0)

        @pl.when(s < NMOE)
        def _moe():
            b = s // (K * NI)
            r = s % (K * NI)
            k = r // NI
            x = hn_ref[0]
            g = jnp.dot(x, wg_ref[0], preferred_element_type=jnp.float32)
            u = jnp.dot(x, wu_ref[0], preferred_element_type=jnp.float32)
            sg = g * (1.0 / (1.0 + jnp.exp(-g)))
            a = sg * u
            o = jnp.dot(a, wd_ref[0], preferred_element_type=jnp.float32)
            w = wts_ref[b, k]

            @pl.when(r == 0)
            def _():
                comb_ref[pl.ds(b, 1)] = (w * o)[None]

            @pl.when(r != 0)
            def _():
                comb_ref[pl.ds(b, 1)] += (w * o)[None]

        @pl.when(s == NMOE)
        def _cast():
            comb16_ref[...] = comb_ref[...].reshape(N, H).astype(jnp.bfloat16)

        @pl.when(s >= NMOE)
        def _logits():
            e16 = emb_ref[...].astype(jnp.bfloat16)
            out_ref[...] = lax.dot_general(
                comb16_ref[...], e16, (((1,), (1,)), ((), ())),
                preferred_element_type=jnp.float32)

    return body


def kernel(input_ids, emb, pos_emb, gamma, beta, Wp, bp, Wr, br, Wg, Wu, Wd):
    B, S = input_ids.shape
    V, H = emb.shape
    R = Wp.shape[0]
    E, _, I = Wg.shape
    K = 2
    N = B * S

    ids_flat = input_ids.reshape(N).astype(jnp.int32)
    tok = _make_sc_gather(V, H, N)(emb, ids_flat)
    tok3 = tok.reshape(B, S, H)

    hn, wts, eids = pl.pallas_call(
        functools.partial(_prep_body, B, S, H, R, E),
        out_shape=(
            jax.ShapeDtypeStruct((B, S, H), jnp.float32),
            jax.ShapeDtypeStruct((B, K), jnp.float32),
            jax.ShapeDtypeStruct((B, K), jnp.int32),
        ),
    )(tok3, pos_emb, gamma, beta, Wp, bp, Wr, br)

    VT = 1280
    NI = 2
    IT = I // NI
    NMOE = B * K * NI
    nsteps = NMOE + V // VT

    def _bki(s):
        sm = jnp.minimum(s, NMOE - 1)
        r = sm % (K * NI)
        return sm // (K * NI), r // NI, r % NI

    def _hn_map(s, ids, wts):
        b, _, _ = _bki(s)
        return (b, 0, 0)

    def _wgu_map(s, ids, wts):
        b, k, i = _bki(s)
        return (ids[b, k], 0, i)

    def _wd_map(s, ids, wts):
        b, k, i = _bki(s)
        return (ids[b, k], i, 0)

    def _emb_map(s, ids, wts):
        return (jnp.maximum(s - NMOE, 0), 0)

    def _out_map(s, ids, wts):
        return (0, jnp.maximum(s - NMOE, 0))

    grid_spec = pltpu.PrefetchScalarGridSpec(
        num_scalar_prefetch=2,
        grid=(nsteps,),
        in_specs=[
            pl.BlockSpec((1, S, H), _hn_map),
            pl.BlockSpec((1, H, IT), _wgu_map),
            pl.BlockSpec((1, H, IT), _wgu_map),
            pl.BlockSpec((1, IT, H), _wd_map),
            pl.BlockSpec((VT, H), _emb_map),
        ],
        out_specs=pl.BlockSpec((N, VT), _out_map),
        scratch_shapes=[pltpu.VMEM((B, S, H), jnp.float32),
                        pltpu.VMEM((N, H), jnp.bfloat16)],
    )
    logits = pl.pallas_call(
        _make_moe_logits_body(B, S, H, I, K, N, NI),
        grid_spec=grid_spec,
        out_shape=jax.ShapeDtypeStruct((N, V), jnp.float32),
    )(eids, wts, hn, Wg, Wu, Wd, emb)

    return logits.reshape(B, S, V)


# speedup vs baseline: 1.0083x; 1.0083x over previous
import functools

import jax
import jax.numpy as jnp
from jax import lax
from jax.experimental import pallas as pl
from jax.experimental.pallas import tpu as pltpu
from jax.experimental.pallas import tpu_sc as plsc


def _make_sc_gather(V, D, N):
    info = plsc.get_sparse_core_info()
    NW = info.num_cores * info.num_subcores
    b_per_w = N // NW
    assert N % NW == 0 and b_per_w % 8 == 0 and D % info.num_lanes == 0
    mesh = plsc.VectorSubcoreMesh(core_axis_name="c", subcore_axis_name="s")

    @functools.partial(
        pl.kernel,
        mesh=mesh,
        out_type=jax.ShapeDtypeStruct((N, D), jnp.float32),
        scratch_types=[
            pltpu.VMEM((b_per_w,), jnp.int32),
            pltpu.VMEM((b_per_w, D), jnp.float32),
            pltpu.SemaphoreType.DMA,
        ],
    )
    def gather_k(table_hbm, idx_hbm, out_hbm, idx_v, rows_v, sem):
        wid = lax.axis_index("s") * info.num_cores + lax.axis_index("c")
        base = wid * b_per_w
        pltpu.sync_copy(idx_hbm.at[pl.ds(base, b_per_w)], idx_v)
        pltpu.async_copy(table_hbm.at[idx_v], rows_v, sem).wait()
        pltpu.sync_copy(rows_v, out_hbm.at[pl.ds(base, b_per_w)])

    return gather_k


def _make_mega_body(B, S, H, I, E, K, N, NI, IT, VT, NMOE):
    def body(tok_ref, pos_ref, g_ref, be_ref, Wp_ref, bp_ref, Wr_ref, br_ref,
             wg_hbm, wu_hbm, wd_hbm, emb_ref, out_ref,
             hn_s, comb_s, pv_s, psm, ids_s, wts_s, wgb, wub, wdb,
             sem_p, sems):
        s = pl.program_id(0)

        def issue(j, slot):
            b = j // (K * NI)
            r = j % (K * NI)
            k = r // NI
            i = r % NI
            e = ids_s[b, k]
            pltpu.make_async_copy(
                wg_hbm.at[e, :, pl.ds(i * IT, IT)], wgb.at[slot],
                sems.at[slot, 0]).start()
            pltpu.make_async_copy(
                wu_hbm.at[e, :, pl.ds(i * IT, IT)], wub.at[slot],
                sems.at[slot, 1]).start()
            pltpu.make_async_copy(
                wd_hbm.at[e, pl.ds(i * IT, IT), :], wdb.at[slot],
                sems.at[slot, 2]).start()

        @pl.when(s == 0)
        def _prep():
            tok = tok_ref[...]
            h = tok + pos_ref[...][None, :, :]
            mu = jnp.mean(h, axis=-1, keepdims=True)
            var = jnp.mean((h - mu) ** 2, axis=-1, keepdims=True)
            hn = (h - mu) * lax.rsqrt(var + 1e-5) * g_ref[...] + be_ref[...]
            hn_s[...] = hn
            pooled = jnp.mean(hn, axis=1)
            rr = lax.dot_general(pooled, Wp_ref[...], (((1,), (1,)), ((), ())),
                                 preferred_element_type=jnp.float32) + bp_ref[...]
            lg = lax.dot_general(rr, Wr_ref[...], (((1,), (1,)), ((), ())),
                                 preferred_element_type=jnp.float32) + br_ref[...]
            m = jnp.max(lg, axis=1, keepdims=True)
            ex = jnp.exp(lg - m)
            p = ex / jnp.sum(ex, axis=1, keepdims=True)
            pv_s[...] = jnp.pad(p, ((0, 8 - B), (0, 128 - E)))
            pltpu.make_async_copy(pv_s, psm, sem_p).start()
            pltpu.make_async_copy(pv_s, psm, sem_p).wait()
            for b in range(B):
                def sel(e, c):
                    m1, j1, m2, j2 = c
                    v = psm[b, e]
                    b1 = v > m1
                    nm1 = jnp.where(b1, v, m1)
                    nj1 = jnp.where(b1, e, j1)
                    c2v = jnp.where(b1, m1, v)
                    c2j = jnp.where(b1, j1, e)
                    b2 = c2v > m2
                    return (nm1, nj1, jnp.where(b2, c2v, m2),
                            jnp.where(b2, c2j, j2))

                m1, j1, m2, j2 = lax.fori_loop(
                    0, E, sel, (-1.0, jnp.int32(0), -1.0, jnp.int32(0)))
                d = m1 + m2 + 1e-8
                ids_s[b, 0] = j1
                ids_s[b, 1] = j2
                wts_s[b, 0] = m1 / d
                wts_s[b, 1] = m2 / d
            issue(0, 0)
            issue(1, 1)

        @pl.when(jnp.logical_and(s >= 1, s <= NMOE))
        def _moe():
            j = s - 1
            slot = lax.rem(j, 2)
            b = j // (K * NI)
            r = j % (K * NI)
            k = r // NI
            for t in range(3):
                pltpu.make_async_copy(
                    wg_hbm.at[0, :, pl.ds(0, IT)] if t != 2
                    else wd_hbm.at[0, pl.ds(0, IT), :],
                    (wgb, wub, wdb)[t].at[slot],
                    sems.at[slot, t]).wait()
            x = hn_s[b]
            g = jnp.dot(x, wgb[slot], preferred_element_type=jnp.float32)
            u = jnp.dot(x, wub[slot], preferred_element_type=jnp.float32)
            a = g * (1.0 / (1.0 + jnp.exp(-g))) * u
            o = jnp.dot(a, wdb[slot], preferred_element_type=jnp.float32)
            w = wts_s[b, k]

            @pl.when(r == 0)
            def _():
                comb_s[pl.ds(b, 1)] = (w * o)[None]

            @pl.when(r != 0)
            def _():
                comb_s[pl.ds(b, 1)] += (w * o)[None]

            @pl.when(j + 2 < NMOE)
            def _():
                issue(j + 2, slot)

        @pl.when(s > NMOE)
        def _logits():
            x2 = comb_s[...].reshape(N, H)
            out_ref[...] = lax.dot_general(
                x2, emb_ref[...], (((1,), (1,)), ((), ())),
                preferred_element_type=jnp.float32)

    return body


def kernel(input_ids, emb, pos_emb, gamma, beta, Wp, bp, Wr, br, Wg, Wu, Wd):
    B, S = input_ids.shape
    V, H = emb.shape
    R = Wp.shape[0]
    E, _, I = Wg.shape
    K = 2
    N = B * S
    VT = 1280
    NI = 2
    IT = I // NI
    NMOE = B * K * NI
    nsteps = 1 + NMOE + V // VT

    ids_flat = input_ids.reshape(N).astype(jnp.int32)
    tok = _make_sc_gather(V, H, N)(emb, ids_flat)
    tok3 = tok.reshape(B, S, H)

    def _emb_map(s):
        return (jnp.maximum(s - NMOE - 1, 0), 0)

    def _out_map(s):
        return (0, jnp.maximum(s - NMOE - 1, 0))

    zero3 = pl.BlockSpec((B, S, H), lambda s: (0, 0, 0))
    logits = pl.pallas_call(
        _make_mega_body(B, S, H, I, E, K, N, NI, IT, VT, NMOE),
        grid=(nsteps,),
        in_specs=[
            zero3,
            pl.BlockSpec((S, H), lambda s: (0, 0)),
            pl.BlockSpec((H,), lambda s: (0,)),
            pl.BlockSpec((H,), lambda s: (0,)),
            pl.BlockSpec((R, H), lambda s: (0, 0)),
            pl.BlockSpec((R,), lambda s: (0,)),
            pl.BlockSpec((E, R), lambda s: (0, 0)),
            pl.BlockSpec((E,), lambda s: (0,)),
            pl.BlockSpec(memory_space=pltpu.MemorySpace.HBM),
            pl.BlockSpec(memory_space=pltpu.MemorySpace.HBM),
            pl.BlockSpec(memory_space=pltpu.MemorySpace.HBM),
            pl.BlockSpec((VT, H), _emb_map),
        ],
        out_specs=pl.BlockSpec((N, VT), _out_map),
        out_shape=jax.ShapeDtypeStruct((N, V), jnp.float32),
        scratch_shapes=[
            pltpu.VMEM((B, S, H), jnp.float32),
            pltpu.VMEM((B, S, H), jnp.float32),
            pltpu.VMEM((8, 128), jnp.float32),
            pltpu.SMEM((8, 128), jnp.float32),
            pltpu.SMEM((B, K), jnp.int32),
            pltpu.SMEM((B, K), jnp.float32),
            pltpu.VMEM((2, H, IT), jnp.float32),
            pltpu.VMEM((2, H, IT), jnp.float32),
            pltpu.VMEM((2, IT, H), jnp.float32),
            pltpu.SemaphoreType.DMA,
            pltpu.SemaphoreType.DMA((2, 3)),
        ],
    )(tok3, pos_emb, gamma, beta, Wp, bp, Wr, br, Wg, Wu, Wd, emb)

    return logits.reshape(B, S, V)


# speedup vs baseline: 1.0227x; 1.0143x over previous
import functools

import jax
import jax.numpy as jnp
from jax import lax
from jax.experimental import pallas as pl
from jax.experimental.pallas import tpu as pltpu
from jax.experimental.pallas import tpu_sc as plsc


def _make_sc_gather(V, D, N):
    info = plsc.get_sparse_core_info()
    NW = info.num_cores * info.num_subcores
    b_per_w = N // NW
    assert N % NW == 0 and b_per_w % 8 == 0 and D % info.num_lanes == 0
    mesh = plsc.VectorSubcoreMesh(core_axis_name="c", subcore_axis_name="s")

    @functools.partial(
        pl.kernel,
        mesh=mesh,
        out_type=jax.ShapeDtypeStruct((N, D), jnp.float32),
        scratch_types=[
            pltpu.VMEM((b_per_w,), jnp.int32),
            pltpu.VMEM((b_per_w, D), jnp.float32),
            pltpu.SemaphoreType.DMA,
        ],
    )
    def gather_k(table_hbm, idx_hbm, out_hbm, idx_v, rows_v, sem):
        wid = lax.axis_index("s") * info.num_cores + lax.axis_index("c")
        base = wid * b_per_w
        pltpu.sync_copy(idx_hbm.at[pl.ds(base, b_per_w)], idx_v)
        pltpu.async_copy(table_hbm.at[idx_v], rows_v, sem).wait()
        pltpu.sync_copy(rows_v, out_hbm.at[pl.ds(base, b_per_w)])

    return gather_k


def _make_mega_body(B, S, H, I, E, K, N, NI, IT, VT, NMOE):
    NP = B * K

    def body(tok_ref, pos_ref, g_ref, be_ref, Wp_ref, bp_ref, Wr_ref, br_ref,
             wg_hbm, wu_hbm, wd_hbm, emb_ref, out_ref,
             hn_s, comb_s, pv_s, psm, se_s, sb_s, sw_s, dup_s, wgb, wub, wdb,
             sem_p, sems):
        s = pl.program_id(0)

        def issue(j, slot):
            p = j // NI
            i = j % NI
            e = se_s[p]
            pltpu.make_async_copy(
                wg_hbm.at[e, :, pl.ds(i * IT, IT)], wgb.at[slot],
                sems.at[slot, 0]).start()
            pltpu.make_async_copy(
                wu_hbm.at[e, :, pl.ds(i * IT, IT)], wub.at[slot],
                sems.at[slot, 1]).start()
            pltpu.make_async_copy(
                wd_hbm.at[e, pl.ds(i * IT, IT), :], wdb.at[slot],
                sems.at[slot, 2]).start()

        @pl.when(s == 0)
        def _prep():
            tok = tok_ref[...]
            h = tok + pos_ref[...][None, :, :]
            mu = jnp.mean(h, axis=-1, keepdims=True)
            var = jnp.mean((h - mu) ** 2, axis=-1, keepdims=True)
            hn = (h - mu) * lax.rsqrt(var + 1e-5) * g_ref[...] + be_ref[...]
            hn_s[...] = hn
            pooled = jnp.mean(hn, axis=1)
            rr = lax.dot_general(pooled, Wp_ref[...], (((1,), (1,)), ((), ())),
                                 preferred_element_type=jnp.float32) + bp_ref[...]
            lg = lax.dot_general(rr, Wr_ref[...], (((1,), (1,)), ((), ())),
                                 preferred_element_type=jnp.float32) + br_ref[...]
            m = jnp.max(lg, axis=1, keepdims=True)
            ex = jnp.exp(lg - m)
            p = ex / jnp.sum(ex, axis=1, keepdims=True)
            pv_s[...] = jnp.pad(p, ((0, 8 - B), (0, 128 - E)))
            comb_s[...] = jnp.zeros((B, S, H), jnp.float32)
            pltpu.make_async_copy(pv_s, psm, sem_p).start()
            pltpu.make_async_copy(pv_s, psm, sem_p).wait()
            items = []
            for b in range(B):
                def sel(e, c):
                    m1, j1, m2, j2 = c
                    v = psm[b, e]
                    b1 = v > m1
                    nm1 = jnp.where(b1, v, m1)
                    nj1 = jnp.where(b1, e, j1)
                    c2v = jnp.where(b1, m1, v)
                    c2j = jnp.where(b1, j1, e)
                    b2 = c2v > m2
                    return (nm1, nj1, jnp.where(b2, c2v, m2),
                            jnp.where(b2, c2j, j2))

                m1, j1, m2, j2 = lax.fori_loop(
                    0, E, sel, (-1.0, jnp.int32(0), -1.0, jnp.int32(0)))
                d = m1 + m2 + 1e-8
                items.append([j1, jnp.int32(b), m1 / d])
                items.append([j2, jnp.int32(b), m2 / d])
            net = [(0, 1), (2, 3), (4, 5), (6, 7),
                   (0, 2), (1, 3), (4, 6), (5, 7),
                   (1, 2), (5, 6),
                   (0, 4), (1, 5), (2, 6), (3, 7),
                   (2, 4), (3, 5),
                   (1, 2), (3, 4), (5, 6)]
            for a, c in net:
                swap = items[a][0] > items[c][0]
                for t in range(3):
                    lo = jnp.where(swap, items[c][t], items[a][t])
                    hi = jnp.where(swap, items[a][t], items[c][t])
                    items[a][t] = lo
                    items[c][t] = hi
            for p in range(NP):
                se_s[p] = items[p][0]
                sb_s[p] = items[p][1]
                sw_s[p] = items[p][2]
                dup_s[p] = (jnp.where(items[p][0] == items[p - 1][0], 1, 0)
                            if p > 0 else jnp.int32(0))
            issue(0, 0)
            issue(1, 1)

        @pl.when(jnp.logical_and(s >= 1, s <= NMOE))
        def _moe():
            j = s - 1
            slot = lax.rem(j, 2)
            p = j // NI
            b = sb_s[p]
            w = sw_s[p]

            @pl.when(dup_s[p] == 0)
            def _wait():
                for t in range(3):
                    pltpu.make_async_copy(
                        wg_hbm.at[0, :, pl.ds(0, IT)] if t != 2
                        else wd_hbm.at[0, pl.ds(0, IT), :],
                        (wgb, wub, wdb)[t].at[slot],
                        sems.at[slot, t]).wait()

            x = hn_s[b]
            g = jnp.dot(x, wgb[slot], preferred_element_type=jnp.float32)
            u = jnp.dot(x, wub[slot], preferred_element_type=jnp.float32)
            a = g * (1.0 / (1.0 + jnp.exp(-g))) * u
            o = jnp.dot(a, wdb[slot], preferred_element_type=jnp.float32)
            comb_s[pl.ds(b, 1)] += (w * o)[None]

            jn = j + 2
            pn = jnp.minimum(jn // NI, NP - 1)

            @pl.when(jnp.logical_and(jn < NMOE, dup_s[pn] == 0))
            def _():
                issue(jn, slot)

        @pl.when(s > NMOE)
        def _logits():
            x2 = comb_s[...].reshape(N, H)
            out_ref[...] = lax.dot_general(
                x2, emb_ref[...], (((1,), (1,)), ((), ())),
                preferred_element_type=jnp.float32)

    return body


def kernel(input_ids, emb, pos_emb, gamma, beta, Wp, bp, Wr, br, Wg, Wu, Wd):
    B, S = input_ids.shape
    V, H = emb.shape
    R = Wp.shape[0]
    E, _, I = Wg.shape
    K = 2
    N = B * S
    VT = 1280
    NI = 2
    IT = I // NI
    NMOE = B * K * NI
    nsteps = 1 + NMOE + V // VT

    ids_flat = input_ids.reshape(N).astype(jnp.int32)
    tok = _make_sc_gather(V, H, N)(emb, ids_flat)
    tok3 = tok.reshape(B, S, H)

    def _emb_map(s):
        return (jnp.maximum(s - NMOE - 1, 0), 0)

    def _out_map(s):
        return (0, jnp.maximum(s - NMOE - 1, 0))

    zero3 = pl.BlockSpec((B, S, H), lambda s: (0, 0, 0))
    logits = pl.pallas_call(
        _make_mega_body(B, S, H, I, E, K, N, NI, IT, VT, NMOE),
        grid=(nsteps,),
        in_specs=[
            zero3,
            pl.BlockSpec((S, H), lambda s: (0, 0)),
            pl.BlockSpec((H,), lambda s: (0,)),
            pl.BlockSpec((H,), lambda s: (0,)),
            pl.BlockSpec((R, H), lambda s: (0, 0)),
            pl.BlockSpec((R,), lambda s: (0,)),
            pl.BlockSpec((E, R), lambda s: (0, 0)),
            pl.BlockSpec((E,), lambda s: (0,)),
            pl.BlockSpec(memory_space=pltpu.MemorySpace.HBM),
            pl.BlockSpec(memory_space=pltpu.MemorySpace.HBM),
            pl.BlockSpec(memory_space=pltpu.MemorySpace.HBM),
            pl.BlockSpec((VT, H), _emb_map),
        ],
        out_specs=pl.BlockSpec((N, VT), _out_map),
        out_shape=jax.ShapeDtypeStruct((N, V), jnp.float32),
        scratch_shapes=[
            pltpu.VMEM((B, S, H), jnp.float32),
            pltpu.VMEM((B, S, H), jnp.float32),
            pltpu.VMEM((8, 128), jnp.float32),
            pltpu.SMEM((8, 128), jnp.float32),
            pltpu.SMEM((B * K,), jnp.int32),
            pltpu.SMEM((B * K,), jnp.int32),
            pltpu.SMEM((B * K,), jnp.float32),
            pltpu.SMEM((B * K,), jnp.int32),
            pltpu.VMEM((2, H, IT), jnp.float32),
            pltpu.VMEM((2, H, IT), jnp.float32),
            pltpu.VMEM((2, IT, H), jnp.float32),
            pltpu.SemaphoreType.DMA,
            pltpu.SemaphoreType.DMA((2, 3)),
        ],
    )(tok3, pos_emb, gamma, beta, Wp, bp, Wr, br, Wg, Wu, Wd, emb)

    return logits.reshape(B, S, V)


# speedup vs baseline: 1.0230x; 1.0002x over previous
import functools

import jax
import jax.numpy as jnp
from jax import lax
from jax.experimental import pallas as pl
from jax.experimental.pallas import tpu as pltpu
from jax.experimental.pallas import tpu_sc as plsc


def _make_sc_gather(V, D, N):
    info = plsc.get_sparse_core_info()
    NW = info.num_cores * info.num_subcores
    b_per_w = N // NW
    assert N % NW == 0 and b_per_w % 8 == 0 and D % info.num_lanes == 0
    mesh = plsc.VectorSubcoreMesh(core_axis_name="c", subcore_axis_name="s")

    @functools.partial(
        pl.kernel,
        mesh=mesh,
        out_type=jax.ShapeDtypeStruct((N, D), jnp.float32),
        scratch_types=[
            pltpu.VMEM((b_per_w,), jnp.int32),
            pltpu.VMEM((b_per_w, D), jnp.float32),
            pltpu.SemaphoreType.DMA,
        ],
    )
    def gather_k(table_hbm, idx_hbm, out_hbm, idx_v, rows_v, sem):
        wid = lax.axis_index("s") * info.num_cores + lax.axis_index("c")
        base = wid * b_per_w
        pltpu.sync_copy(idx_hbm.at[pl.ds(base, b_per_w)], idx_v)
        pltpu.async_copy(table_hbm.at[idx_v], rows_v, sem).wait()
        pltpu.sync_copy(rows_v, out_hbm.at[pl.ds(base, b_per_w)])

    return gather_k


def _make_mega_body(B, S, H, I, E, K, N, NI, IT, VT, NMOE):
    NP = B * K

    def body(tok_ref, pos_ref, g_ref, be_ref, Wp_ref, bp_ref, Wr_ref, br_ref,
             wg_hbm, wu_hbm, wd_hbm, emb_ref, out_ref,
             hn_s, comb_s, pv_s, psm, se_s, sb_s, sw_s, dup_s, wgb, wub, wdb,
             sem_p, sems):
        s = pl.program_id(0)

        def issue(j, slot):
            p = j // NI
            i = j % NI
            e = se_s[p]
            pltpu.make_async_copy(
                wg_hbm.at[e, :, pl.ds(i * IT, IT)], wgb.at[slot],
                sems.at[slot, 0]).start()
            pltpu.make_async_copy(
                wu_hbm.at[e, :, pl.ds(i * IT, IT)], wub.at[slot],
                sems.at[slot, 1]).start()
            pltpu.make_async_copy(
                wd_hbm.at[e, pl.ds(i * IT, IT), :], wdb.at[slot],
                sems.at[slot, 2]).start()

        @pl.when(s == 0)
        def _prep():
            tok = tok_ref[...]
            h = tok + pos_ref[...][None, :, :]
            mu = jnp.mean(h, axis=-1, keepdims=True)
            var = jnp.mean((h - mu) ** 2, axis=-1, keepdims=True)
            hn = (h - mu) * lax.rsqrt(var + 1e-5) * g_ref[...] + be_ref[...]
            pooled = jnp.mean(hn, axis=1)
            rr = lax.dot_general(pooled, Wp_ref[...], (((1,), (1,)), ((), ())),
                                 preferred_element_type=jnp.float32) + bp_ref[...]
            lg = lax.dot_general(rr, Wr_ref[...], (((1,), (1,)), ((), ())),
                                 preferred_element_type=jnp.float32) + br_ref[...]
            m = jnp.max(lg, axis=1, keepdims=True)
            ex = jnp.exp(lg - m)
            p = ex / jnp.sum(ex, axis=1, keepdims=True)
            pv_s[...] = jnp.pad(p, ((0, 8 - B), (0, 128 - E)))
            pltpu.make_async_copy(pv_s, psm, sem_p).start()
            pltpu.make_async_copy(pv_s, psm, sem_p).wait()
            items = []
            for b in range(B):
                def sel(e, c):
                    m1, j1, m2, j2 = c
                    v = psm[b, e]
                    b1 = v > m1
                    nm1 = jnp.where(b1, v, m1)
                    nj1 = jnp.where(b1, e, j1)
                    c2v = jnp.where(b1, m1, v)
                    c2j = jnp.where(b1, j1, e)
                    b2 = c2v > m2
                    return (nm1, nj1, jnp.where(b2, c2v, m2),
                            jnp.where(b2, c2j, j2))

                m1, j1, m2, j2 = lax.fori_loop(
                    0, E, sel, (-1.0, jnp.int32(0), -1.0, jnp.int32(0)))
                d = m1 + m2 + 1e-8
                items.append([j1, jnp.int32(b), m1 / d])
                items.append([j2, jnp.int32(b), m2 / d])
            net = [(0, 1), (2, 3), (4, 5), (6, 7),
                   (0, 2), (1, 3), (4, 6), (5, 7),
                   (1, 2), (5, 6),
                   (0, 4), (1, 5), (2, 6), (3, 7),
                   (2, 4), (3, 5),
                   (1, 2), (3, 4), (5, 6)]
            for a, c in net:
                swap = items[a][0] > items[c][0]
                for t in range(3):
                    lo = jnp.where(swap, items[c][t], items[a][t])
                    hi = jnp.where(swap, items[a][t], items[c][t])
                    items[a][t] = lo
                    items[c][t] = hi
            for p in range(NP):
                se_s[p] = items[p][0]
                sb_s[p] = items[p][1]
                sw_s[p] = items[p][2]
                dup_s[p] = (jnp.where(items[p][0] == items[p - 1][0], 1, 0)
                            if p > 0 else jnp.int32(0))
            issue(0, 0)
            issue(1, 1)
            hn_s[...] = hn
            comb_s[...] = jnp.zeros((B, S, H), jnp.float32)

        @pl.when(jnp.logical_and(s >= 1, s <= NMOE))
        def _moe():
            j = s - 1
            slot = lax.rem(j, 2)
            p = j // NI
            b = sb_s[p]
            w = sw_s[p]

            @pl.when(dup_s[p] == 0)
            def _wait():
                for t in range(3):
                    pltpu.make_async_copy(
                        wg_hbm.at[0, :, pl.ds(0, IT)] if t != 2
                        else wd_hbm.at[0, pl.ds(0, IT), :],
                        (wgb, wub, wdb)[t].at[slot],
                        sems.at[slot, t]).wait()

            x = hn_s[b]
            g = jnp.dot(x, wgb[slot], preferred_element_type=jnp.float32)
            u = jnp.dot(x, wub[slot], preferred_element_type=jnp.float32)
            a = g * (1.0 / (1.0 + jnp.exp(-g))) * u
            o = jnp.dot(a, wdb[slot], preferred_element_type=jnp.float32)
            comb_s[pl.ds(b, 1)] += (w * o)[None]

            jn = j + 2
            pn = jnp.minimum(jn // NI, NP - 1)

            @pl.when(jnp.logical_and(jn < NMOE, dup_s[pn] == 0))
            def _():
                issue(jn, slot)

        @pl.when(s > NMOE)
        def _logits():
            x2 = comb_s[...].reshape(N, H)
            out_ref[...] = lax.dot_general(
                x2, emb_ref[...], (((1,), (1,)), ((), ())),
                preferred_element_type=jnp.float32)

    return body


def kernel(input_ids, emb, pos_emb, gamma, beta, Wp, bp, Wr, br, Wg, Wu, Wd):
    B, S = input_ids.shape
    V, H = emb.shape
    R = Wp.shape[0]
    E, _, I = Wg.shape
    K = 2
    N = B * S
    VT = 1280
    NI = 2
    IT = I // NI
    NMOE = B * K * NI
    nsteps = 1 + NMOE + V // VT

    ids_flat = input_ids.reshape(N).astype(jnp.int32)
    tok = _make_sc_gather(V, H, N)(emb, ids_flat)
    tok3 = tok.reshape(B, S, H)

    def _emb_map(s):
        return (jnp.maximum(s - NMOE - 1, 0), 0)

    def _out_map(s):
        return (0, jnp.maximum(s - NMOE - 1, 0))

    zero3 = pl.BlockSpec((B, S, H), lambda s: (0, 0, 0))
    logits = pl.pallas_call(
        _make_mega_body(B, S, H, I, E, K, N, NI, IT, VT, NMOE),
        grid=(nsteps,),
        in_specs=[
            zero3,
            pl.BlockSpec((S, H), lambda s: (0, 0)),
            pl.BlockSpec((H,), lambda s: (0,)),
            pl.BlockSpec((H,), lambda s: (0,)),
            pl.BlockSpec((R, H), lambda s: (0, 0)),
            pl.BlockSpec((R,), lambda s: (0,)),
            pl.BlockSpec((E, R), lambda s: (0, 0)),
            pl.BlockSpec((E,), lambda s: (0,)),
            pl.BlockSpec(memory_space=pltpu.MemorySpace.HBM),
            pl.BlockSpec(memory_space=pltpu.MemorySpace.HBM),
            pl.BlockSpec(memory_space=pltpu.MemorySpace.HBM),
            pl.BlockSpec((VT, H), _emb_map),
        ],
        out_specs=pl.BlockSpec((N, VT), _out_map),
        out_shape=jax.ShapeDtypeStruct((N, V), jnp.float32),
        scratch_shapes=[
            pltpu.VMEM((B, S, H), jnp.float32),
            pltpu.VMEM((B, S, H), jnp.float32),
            pltpu.VMEM((8, 128), jnp.float32),
            pltpu.SMEM((8, 128), jnp.float32),
            pltpu.SMEM((B * K,), jnp.int32),
            pltpu.SMEM((B * K,), jnp.int32),
            pltpu.SMEM((B * K,), jnp.float32),
            pltpu.SMEM((B * K,), jnp.int32),
            pltpu.VMEM((2, H, IT), jnp.float32),
            pltpu.VMEM((2, H, IT), jnp.float32),
            pltpu.VMEM((2, IT, H), jnp.float32),
            pltpu.SemaphoreType.DMA,
            pltpu.SemaphoreType.DMA((2, 3)),
        ],
    )(tok3, pos_emb, gamma, beta, Wp, bp, Wr, br, Wg, Wu, Wd, emb)

    return logits.reshape(B, S, V)
